# baseline (speedup 1.0000x reference)
import functools

import jax
import jax.numpy as jnp
from jax import lax
from jax.experimental import pallas as pl
from jax.experimental.pallas import tpu as pltpu
from jax.experimental.pallas import tpu_sc as plsc

N = 10000
NPAD = 10240
C = 64
CW = 128
K = 10
NTILES = 16
ROWS = NPAD // NTILES
E = 320000
EPT = E // NTILES
W = 128
EPT_PAD = 20480
NCHUNK = EPT_PAD // W
NRC = ROWS // W


def _mlp_body(f_ref, w1_ref, b1_ref, w2_ref, b2_ref, o_ref):
    h = jnp.dot(f_ref[...], w1_ref[...],
                preferred_element_type=jnp.float32) + b1_ref[...]
    h = jnp.maximum(h, 0.0)
    x = jnp.dot(h, w2_ref[...],
                preferred_element_type=jnp.float32) + b2_ref[...]
    o_ref[...] = jnp.concatenate(
        [x, jnp.zeros((x.shape[0], CW - C), jnp.float32)], axis=1)


def _mlp(feature_pad, W1, b1, W2, b2):
    nblk = NPAD // 1024
    return pl.pallas_call(
        _mlp_body,
        grid=(nblk,),
        in_specs=[
            pl.BlockSpec((1024, 128), lambda i: (i, 0)),
            pl.BlockSpec((128, 128), lambda i: (0, 0)),
            pl.BlockSpec((1, 128), lambda i: (0, 0)),
            pl.BlockSpec((128, C), lambda i: (0, 0)),
            pl.BlockSpec((1, C), lambda i: (0, 0)),
        ],
        out_specs=pl.BlockSpec((1024, CW), lambda i: (i, 0)),
        out_shape=jax.ShapeDtypeStruct((NPAD, CW), jnp.float32),
    )(feature_pad, W1, b1.reshape(1, 128), W2, b2.reshape(1, C))


def _red_body(a_ref, b_ref, h_ref, o_ref):
    j = pl.program_id(1)

    @pl.when(j == 0)
    def _():
        o_ref[...] = a_ref[...][:, :C] + b_ref[...][:, :C]

    @pl.when(j > 0)
    def _():
        o_ref[...] = o_ref[...] + h_ref[0][:, :C]


def _final_reduce(a, b, hws):
    nflat = hws.shape[0]
    return pl.pallas_call(
        _red_body,
        grid=(10, nflat + 1),
        in_specs=[
            pl.BlockSpec((1000, CW), lambda i, j: (i, 0)),
            pl.BlockSpec((1000, CW), lambda i, j: (i, 0)),
            pl.BlockSpec((1, 1000, CW),
                         lambda i, j: (jnp.maximum(j - 1, 0), i, 0)),
        ],
        out_specs=pl.BlockSpec((1000, C), lambda i, j: (i, 0)),
        out_shape=jax.ShapeDtypeStruct((N, C), jnp.float32),
    )(a, b, hws)


def _splat(vec16, idx):
    idxs = jnp.broadcast_to(jnp.asarray(idx, jnp.int32), (16,))
    return jnp.take_along_axis(vec16, idxs, axis=0)


def _prop_body(xw_hbm, edges_hbm, temps_hbm,
               out_hbm, hw_hbm,
               hnext, ebufA, ebufB, dstidx, msgA, msgB, temps_v,
               semGA, semGB, semSA, semSB, semEA, semEB):
    cid = lax.axis_index("c")
    sid = lax.axis_index("s")
    rbase = sid * ROWS

    pltpu.sync_copy(temps_hbm.at[cid], temps_v)
    t16 = temps_v[...]
    t0 = _splat(t16, 0)

    for i in range(NRC):
        rs = pl.ds(rbase + i * W, W)
        pltpu.sync_copy(xw_hbm.at[rs], msgA)

        def _s0(r, _):
            for j in range(C // 16):
                sl = pl.ds(j * 16, 16)
                msgA[r, sl] = msgA[r, sl] * t0
            return 0
        lax.fori_loop(0, W, _s0, 0)
        pltpu.sync_copy(msgA, hw_hbm.at[cid, 0, rs])
        pltpu.sync_copy(msgA, out_hbm.at[cid, rs])

    def _scale(mref, eref, rk):
        def _q(q, _):
            n16 = lax.bitcast_convert_type(
                eref[2, pl.ds(q * 16, 16)], jnp.float32) * rk
            for e in range(16):
                nb = jnp.broadcast_to(
                    lax.slice(n16, (e,), (e + 1,)), (16,))
                row = q * 16 + e
                for j in range(C // 16):
                    sl = pl.ds(j * 16, 16)
                    mref[row, sl] = mref[row, sl] * nb
            return 0
        lax.fori_loop(0, W // 16, _q, 0)

    def _cp_dstidx(eref, row):
        for j in range(W // 16):
            sl = pl.ds(j * 16, 16)
            dstidx[row, sl] = eref[1, sl]

    eblk = edges_hbm.at[cid, sid]
    NPAIR = NCHUNK // 2

    def _hop(k, _):
        def _z(r, _):
            for j in range(CW // 16):
                msgA[r, pl.ds(j * 16, 16)] = jnp.zeros((16,), jnp.float32)
            return 0
        lax.fori_loop(0, W, _z, 0)
        for i in range(NRC):
            pltpu.sync_copy(msgA, hnext.at[pl.ds(rbase + i * W, W)])
        plsc.subcore_barrier()

        rk = _splat(t16, k + 1) / _splat(t16, k)
        hsrc = hw_hbm.at[cid, k]

        pltpu.sync_copy(eblk.at[0], ebufA)
        pltpu.async_copy(hsrc.at[ebufA.at[0]], msgA, semGA)
        pltpu.async_copy(eblk.at[1], ebufB, semEB)

        def _pair(m, _):
            c0 = 2 * m
            pltpu.make_async_copy(eblk.at[0], ebufB, semEB).wait()

            @pl.when(m > 0)
            def _():
                pltpu.make_async_copy(
                    msgB, hnext.at[dstidx.at[1]], semSB).wait()
            pltpu.async_copy(hsrc.at[ebufB.at[0]], msgB, semGB)
            pltpu.make_async_copy(hsrc.at[ebufA.at[0]], msgA, semGA).wait()
            _scale(msgA, ebufA, rk)
            _cp_dstidx(ebufA, 0)
            pltpu.async_copy(msgA, hnext.at[dstidx.at[0]], semSA, add=True)
            c2 = jnp.minimum(c0 + 2, NCHUNK - 1)
            pltpu.async_copy(eblk.at[c2], ebufA, semEA)
            pltpu.make_async_copy(hsrc.at[ebufB.at[0]], msgB, semGB).wait()
            _scale(msgB, ebufB, rk)
            _cp_dstidx(ebufB, 1)
            pltpu.async_copy(msgB, hnext.at[dstidx.at[1]], semSB, add=True)
            c3 = jnp.minimum(c0 + 3, NCHUNK - 1)
            pltpu.async_copy(eblk.at[c3], ebufB, semEB)
            pltpu.make_async_copy(eblk.at[0], ebufA, semEA).wait()
            pltpu.make_async_copy(
                msgA, hnext.at[dstidx.at[0]], semSA).wait()
            pltpu.async_copy(hsrc.at[ebufA.at[0]], msgA, semGA)
            return 0
        lax.fori_loop(0, NPAIR, _pair, 0)
        pltpu.make_async_copy(hsrc.at[ebufA.at[0]], msgA, semGA).wait()
        pltpu.make_async_copy(msgB, hnext.at[dstidx.at[1]], semSB).wait()
        pltpu.make_async_copy(eblk.at[0], ebufB, semEB).wait()
        plsc.subcore_barrier()

        for i in range(NRC):
            rs = pl.ds(rbase + i * W, W)
            pltpu.sync_copy(hnext.at[rs], msgA)
            pltpu.sync_copy(msgA, hw_hbm.at[cid, k + 1, rs])
        return 0

    lax.fori_loop(0, K, _hop, 0)


@functools.lru_cache(maxsize=None)
def _get_prop():
  return pl.kernel(
    _prop_body,
    out_type=(
        jax.ShapeDtypeStruct((2, NPAD, CW), jnp.float32),
        jax.ShapeDtypeStruct((2, K + 1, NPAD, CW), jnp.float32),
    ),
    mesh=plsc.VectorSubcoreMesh(core_axis_name="c", subcore_axis_name="s",
                                num_cores=2, num_subcores=NTILES),
    scratch_types=[
        pltpu.VMEM_SHARED((NPAD, CW), jnp.float32),
        pltpu.VMEM((3, W), jnp.int32),
        pltpu.VMEM((3, W), jnp.int32),
        pltpu.VMEM((2, W), jnp.int32),
        pltpu.VMEM((W, CW), jnp.float32),
        pltpu.VMEM((W, CW), jnp.float32),
        pltpu.VMEM((16,), jnp.float32),
        pltpu.SemaphoreType.DMA,
        pltpu.SemaphoreType.DMA,
        pltpu.SemaphoreType.DMA,
        pltpu.SemaphoreType.DMA,
        pltpu.SemaphoreType.DMA,
        pltpu.SemaphoreType.DMA,
    ],
  )


def _prep_edges(ei, nrm):
    src = ei[0].astype(jnp.int32).reshape(NTILES, EPT)
    dst = ei[1].astype(jnp.int32).reshape(NTILES, EPT)
    nr = nrm.astype(jnp.float32).reshape(NTILES, EPT)
    pad = EPT_PAD - EPT
    psrc = jnp.full((NTILES, pad), N - 2, jnp.int32)
    pdst = jnp.broadcast_to(N + (jnp.arange(pad, dtype=jnp.int32) % 240),
                            (NTILES, pad))
    pnrm = jnp.zeros((NTILES, pad), jnp.float32)
    src = jnp.concatenate([src, psrc], 1).reshape(NTILES, NCHUNK, W)
    dst = jnp.concatenate([dst, pdst], 1).reshape(NTILES, NCHUNK, W)
    nr = lax.bitcast_convert_type(
        jnp.concatenate([nr, pnrm], 1).reshape(NTILES, NCHUNK, W), jnp.int32)
    return jnp.stack([src, dst, nr], axis=2)


def kernel(feature, edge_index, edge_index2, norm_A, norm_A_2,
           W1, b1, W2, b2, temp1, temp2):
    feature_pad = jnp.pad(feature, ((0, NPAD - N), (0, 0)))
    x = _mlp(feature_pad, W1, b1, W2, b2)

    e1 = _prep_edges(edge_index, norm_A)
    e2 = _prep_edges(edge_index2, norm_A_2)
    edges = jnp.stack([e1, e2])
    temps = jnp.stack([jnp.pad(temp1, (0, 5)), jnp.pad(temp2, (0, 5))])

    out, hw = _get_prop()(x, edges, temps)
    hws = hw[:, 1:].reshape(2 * K, NPAD, CW)[:, :N]
    return _final_reduce(out[0, :N], out[1, :N], hws)


# speedup vs baseline: 1.1638x; 1.1638x over previous
import functools

import jax
import jax.numpy as jnp
from jax import lax
from jax.experimental import pallas as pl
from jax.experimental.pallas import tpu as pltpu
from jax.experimental.pallas import tpu_sc as plsc

N = 10000
NPAD = 10240
C = 64
CW = 128
K = 10
NTILES = 16
ROWS = NPAD // NTILES
E = 320000
EPT = E // NTILES
W = 128
EPT_PAD = 20480
NCHUNK = EPT_PAD // W
NRC = ROWS // W


def _mlp_body(f_ref, w1_ref, b1_ref, w2_ref, b2_ref, o_ref):
    h = jnp.dot(f_ref[...], w1_ref[...],
                preferred_element_type=jnp.float32) + b1_ref[...]
    h = jnp.maximum(h, 0.0)
    x = jnp.dot(h, w2_ref[...],
                preferred_element_type=jnp.float32) + b2_ref[...]
    o_ref[...] = jnp.concatenate(
        [x, jnp.zeros((x.shape[0], CW - C), jnp.float32)], axis=1)


def _mlp(feature_pad, W1, b1, W2, b2):
    nblk = NPAD // 1024
    return pl.pallas_call(
        _mlp_body,
        grid=(nblk,),
        in_specs=[
            pl.BlockSpec((1024, 128), lambda i: (i, 0)),
            pl.BlockSpec((128, 128), lambda i: (0, 0)),
            pl.BlockSpec((1, 128), lambda i: (0, 0)),
            pl.BlockSpec((128, C), lambda i: (0, 0)),
            pl.BlockSpec((1, C), lambda i: (0, 0)),
        ],
        out_specs=pl.BlockSpec((1024, CW), lambda i: (i, 0)),
        out_shape=jax.ShapeDtypeStruct((NPAD, CW), jnp.float32),
    )(feature_pad, W1, b1.reshape(1, 128), W2, b2.reshape(1, C))


def _add_body(a_ref, b_ref, o_ref):
    o_ref[...] = a_ref[...] + b_ref[...]


def _final_add(a, b):
    return pl.pallas_call(
        _add_body,
        grid=(10,),
        in_specs=[
            pl.BlockSpec((1000, C), lambda i: (i, 0)),
            pl.BlockSpec((1000, C), lambda i: (i, 0)),
        ],
        out_specs=pl.BlockSpec((1000, C), lambda i: (i, 0)),
        out_shape=jax.ShapeDtypeStruct((N, C), jnp.float32),
    )(a, b)


def _splat(vec16, idx):
    idxs = jnp.broadcast_to(jnp.asarray(idx, jnp.int32), (16,))
    return jnp.take_along_axis(vec16, idxs, axis=0)


def _prop_body(xw_hbm, edges_hbm, temps_hbm,
               out_hbm, hw_hbm,
               hnext, ebufA, ebufB, dstidx, msgA, msgB, temps_v,
               semGA, semGB, semSA, semSB):
    cid = lax.axis_index("c")
    sid = lax.axis_index("s")
    rbase = sid * ROWS

    pltpu.sync_copy(temps_hbm.at[cid], temps_v)
    t16 = temps_v[...]
    t0 = _splat(t16, 0)

    for i in range(NRC):
        rs = pl.ds(rbase + i * W, W)
        pltpu.sync_copy(xw_hbm.at[rs], msgA)

        def _s0(r, _):
            for j in range(C // 16):
                sl = pl.ds(j * 16, 16)
                msgA[r, sl] = msgA[r, sl] * t0
            return 0
        lax.fori_loop(0, W, _s0, 0)
        pltpu.sync_copy(msgA, hw_hbm.at[cid, 1, rs])
        pltpu.sync_copy(msgA, out_hbm.at[cid, rs])

    def _scale(mref, eref, rk):
        def _q(q, _):
            n16 = lax.bitcast_convert_type(
                eref[2, pl.ds(q * 16, 16)], jnp.float32) * rk
            for e in range(16):
                nb = jnp.broadcast_to(
                    lax.slice(n16, (e,), (e + 1,)), (16,))
                row = q * 16 + e
                for j in range(C // 16):
                    sl = pl.ds(j * 16, 16)
                    mref[row, sl] = mref[row, sl] * nb
            return 0
        lax.fori_loop(0, W // 16, _q, 0)

    def _cp_dstidx(eref, row):
        for j in range(W // 16):
            sl = pl.ds(j * 16, 16)
            dstidx[row, sl] = eref[1, sl]

    def _hop(bsrc, bdst, k):
        def _z(r, _):
            for j in range(CW // 16):
                sl = pl.ds(j * 16, 16)
                msgA[r, sl] = jnp.zeros((16,), jnp.float32)
                msgB[r, sl] = jnp.zeros((16,), jnp.float32)
            return 0
        lax.fori_loop(0, W, _z, 0)
        for i in range(NRC):
            pltpu.sync_copy(msgA, hnext.at[pl.ds(rbase + i * W, W)])
        plsc.subcore_barrier()

        rk = _splat(t16, k + 1) / _splat(t16, k)
        hsrc = hw_hbm.at[cid, bsrc]

        pltpu.sync_copy(edges_hbm.at[cid, sid, 0], ebufA)
        pltpu.async_copy(hsrc.at[ebufA.at[0]], msgA, semGA)
        _cp_dstidx(ebufA, 1)
        pltpu.async_copy(msgB, hnext.at[dstidx.at[1]], semSB, add=True)

        def _pair(m, _):
            c0 = 2 * m
            pltpu.sync_copy(edges_hbm.at[cid, sid, c0 + 1], ebufB)
            pltpu.make_async_copy(msgB, hnext.at[dstidx.at[1]], semSB).wait()
            pltpu.async_copy(hsrc.at[ebufB.at[0]], msgB, semGB)
            pltpu.make_async_copy(hsrc.at[ebufA.at[0]], msgA, semGA).wait()
            _scale(msgA, ebufA, rk)
            _cp_dstidx(ebufA, 0)
            pltpu.async_copy(msgA, hnext.at[dstidx.at[0]], semSA, add=True)
            c2 = jnp.minimum(c0 + 2, NCHUNK - 1)
            pltpu.sync_copy(edges_hbm.at[cid, sid, c2], ebufA)
            pltpu.make_async_copy(msgA, hnext.at[dstidx.at[0]], semSA).wait()
            pltpu.async_copy(hsrc.at[ebufA.at[0]], msgA, semGA)
            pltpu.make_async_copy(hsrc.at[ebufB.at[0]], msgB, semGB).wait()
            _scale(msgB, ebufB, rk)
            _cp_dstidx(ebufB, 1)
            pltpu.async_copy(msgB, hnext.at[dstidx.at[1]], semSB, add=True)
            return 0
        lax.fori_loop(0, NCHUNK // 2, _pair, 0)
        pltpu.make_async_copy(hsrc.at[ebufA.at[0]], msgA, semGA).wait()
        pltpu.make_async_copy(msgB, hnext.at[dstidx.at[1]], semSB).wait()
        plsc.subcore_barrier()

        for i in range(NRC):
            rs = pl.ds(rbase + i * W, W)
            pltpu.sync_copy(hnext.at[rs], msgA)
            pltpu.sync_copy(msgA, hw_hbm.at[cid, bdst, rs])
            pltpu.sync_copy(out_hbm.at[cid, rs], msgB)

            def _a(r, _):
                for j in range(C // 16):
                    sl = pl.ds(j * 16, 16)
                    msgB[r, sl] = msgB[r, sl] + msgA[r, sl]
                return 0
            lax.fori_loop(0, W, _a, 0)
            pltpu.sync_copy(msgB, out_hbm.at[cid, rs])

    def _pair(it, _):
        _hop(1, 0, 2 * it)
        _hop(0, 1, 2 * it + 1)
        return 0
    lax.fori_loop(0, K // 2, _pair, 0)


@functools.lru_cache(maxsize=None)
def _get_prop():
  return pl.kernel(
    _prop_body,
    out_type=(
        jax.ShapeDtypeStruct((2, NPAD, CW), jnp.float32),
        jax.ShapeDtypeStruct((2, 2, NPAD, CW), jnp.float32),
    ),
    mesh=plsc.VectorSubcoreMesh(core_axis_name="c", subcore_axis_name="s",
                                num_cores=2, num_subcores=NTILES),
    scratch_types=[
        pltpu.VMEM_SHARED((NPAD, CW), jnp.float32),
        pltpu.VMEM((3, W), jnp.int32),
        pltpu.VMEM((3, W), jnp.int32),
        pltpu.VMEM((2, W), jnp.int32),
        pltpu.VMEM((W, CW), jnp.float32),
        pltpu.VMEM((W, CW), jnp.float32),
        pltpu.VMEM((16,), jnp.float32),
        pltpu.SemaphoreType.DMA,
        pltpu.SemaphoreType.DMA,
        pltpu.SemaphoreType.DMA,
        pltpu.SemaphoreType.DMA,
    ],
  )


def _prep_edges(ei, nrm):
    src = ei[0].astype(jnp.int32).reshape(NTILES, EPT)
    dst = ei[1].astype(jnp.int32).reshape(NTILES, EPT)
    nr = nrm.astype(jnp.float32).reshape(NTILES, EPT)
    pad = EPT_PAD - EPT
    psrc = jnp.full((NTILES, pad), N - 2, jnp.int32)
    pdst = jnp.broadcast_to(N + (jnp.arange(pad, dtype=jnp.int32) % 240),
                            (NTILES, pad))
    pnrm = jnp.zeros((NTILES, pad), jnp.float32)
    src = jnp.concatenate([src, psrc], 1).reshape(NTILES, NCHUNK, W)
    dst = jnp.concatenate([dst, pdst], 1).reshape(NTILES, NCHUNK, W)
    nr = lax.bitcast_convert_type(
        jnp.concatenate([nr, pnrm], 1).reshape(NTILES, NCHUNK, W), jnp.int32)
    return jnp.stack([src, dst, nr], axis=2)


def kernel(feature, edge_index, edge_index2, norm_A, norm_A_2,
           W1, b1, W2, b2, temp1, temp2):
    feature_pad = jnp.pad(feature, ((0, NPAD - N), (0, 0)))
    x = _mlp(feature_pad, W1, b1, W2, b2)

    e1 = _prep_edges(edge_index, norm_A)
    e2 = _prep_edges(edge_index2, norm_A_2)
    edges = jnp.stack([e1, e2])
    temps = jnp.stack([jnp.pad(temp1, (0, 5)), jnp.pad(temp2, (0, 5))])

    out, _ = _get_prop()(x, edges, temps)
    return _final_add(out[0, :N, :C], out[1, :N, :C])


# speedup vs baseline: 1.1640x; 1.0002x over previous
import functools

import jax
import jax.numpy as jnp
from jax import lax
from jax.experimental import pallas as pl
from jax.experimental.pallas import tpu as pltpu
from jax.experimental.pallas import tpu_sc as plsc

N = 10000
NPAD = 10240
C = 64
CW = 128
K = 10
NTILES = 16
ROWS = NPAD // NTILES
E = 320000
EPT = E // NTILES
W = 128
EPT_PAD = 20480
NCHUNK = EPT_PAD // W
NRC = ROWS // W


def _mlp_body(f_ref, w1_ref, b1_ref, w2_ref, b2_ref, o_ref):
    h = jnp.dot(f_ref[...], w1_ref[...],
                preferred_element_type=jnp.float32) + b1_ref[...]
    h = jnp.maximum(h, 0.0)
    x = jnp.dot(h, w2_ref[...],
                preferred_element_type=jnp.float32) + b2_ref[...]
    o_ref[...] = jnp.concatenate(
        [x, jnp.zeros((x.shape[0], CW - C), jnp.float32)], axis=1)


def _mlp(feature_pad, W1, b1, W2, b2):
    nblk = NPAD // 1024
    return pl.pallas_call(
        _mlp_body,
        grid=(nblk,),
        in_specs=[
            pl.BlockSpec((1024, 128), lambda i: (i, 0)),
            pl.BlockSpec((128, 128), lambda i: (0, 0)),
            pl.BlockSpec((1, 128), lambda i: (0, 0)),
            pl.BlockSpec((128, C), lambda i: (0, 0)),
            pl.BlockSpec((1, C), lambda i: (0, 0)),
        ],
        out_specs=pl.BlockSpec((1024, CW), lambda i: (i, 0)),
        out_shape=jax.ShapeDtypeStruct((NPAD, CW), jnp.float32),
    )(feature_pad, W1, b1.reshape(1, 128), W2, b2.reshape(1, C))


def _add_body(a_ref, b_ref, o_ref):
    o_ref[...] = a_ref[...] + b_ref[...]


def _final_add(a, b):
    return pl.pallas_call(
        _add_body,
        grid=(10,),
        in_specs=[
            pl.BlockSpec((1000, C), lambda i: (i, 0)),
            pl.BlockSpec((1000, C), lambda i: (i, 0)),
        ],
        out_specs=pl.BlockSpec((1000, C), lambda i: (i, 0)),
        out_shape=jax.ShapeDtypeStruct((N, C), jnp.float32),
    )(a, b)


def _splat(vec16, idx):
    idxs = jnp.broadcast_to(jnp.asarray(idx, jnp.int32), (16,))
    return jnp.take_along_axis(vec16, idxs, axis=0)


def _prop_body(xw_hbm, edges_hbm, temps_hbm,
               out_hbm, hw_hbm,
               hnext, ebufA, ebufB, dstidx, msgA, msgB, temps_v,
               semGA, semGB, semSA, semSB, semEA, semEB):
    cid = lax.axis_index("c")
    sid = lax.axis_index("s")
    rbase = sid * ROWS

    pltpu.sync_copy(temps_hbm.at[cid], temps_v)
    t16 = temps_v[...]
    t0 = _splat(t16, 0)

    for i in range(NRC):
        rs = pl.ds(rbase + i * W, W)
        pltpu.sync_copy(xw_hbm.at[rs], msgA)

        def _s0(r, _):
            for j in range(C // 16):
                sl = pl.ds(j * 16, 16)
                msgA[r, sl] = msgA[r, sl] * t0
            return 0
        lax.fori_loop(0, W, _s0, 0)
        pltpu.sync_copy(msgA, hw_hbm.at[cid, 1, rs])
        pltpu.sync_copy(msgA, out_hbm.at[cid, rs])

    def _scale(mref, eref, rk):
        def _q(q, _):
            n16 = lax.bitcast_convert_type(
                eref[2, pl.ds(q * 16, 16)], jnp.float32) * rk
            for e in range(16):
                nb = jnp.broadcast_to(
                    lax.slice(n16, (e,), (e + 1,)), (16,))
                row = q * 16 + e
                for j in range(C // 16):
                    sl = pl.ds(j * 16, 16)
                    mref[row, sl] = mref[row, sl] * nb
            return 0
        lax.fori_loop(0, W // 16, _q, 0)

    def _cp_dstidx(eref, row):
        for j in range(W // 16):
            sl = pl.ds(j * 16, 16)
            dstidx[row, sl] = eref[1, sl]

    def _hop(bsrc, bdst, k):
        def _z(r, _):
            for j in range(CW // 16):
                sl = pl.ds(j * 16, 16)
                msgA[r, sl] = jnp.zeros((16,), jnp.float32)
                msgB[r, sl] = jnp.zeros((16,), jnp.float32)
            return 0
        lax.fori_loop(0, W, _z, 0)
        for i in range(NRC):
            pltpu.sync_copy(msgA, hnext.at[pl.ds(rbase + i * W, W)])
        plsc.subcore_barrier()

        rk = _splat(t16, k + 1) / _splat(t16, k)
        hsrc = hw_hbm.at[cid, bsrc]

        eblk = edges_hbm.at[cid, sid]
        pltpu.sync_copy(eblk.at[0], ebufA)
        pltpu.async_copy(hsrc.at[ebufA.at[0]], msgA, semGA)
        pltpu.async_copy(eblk.at[1], ebufB, semEB)
        _cp_dstidx(ebufA, 1)
        pltpu.async_copy(msgB, hnext.at[dstidx.at[1]], semSB, add=True)

        def _pair(m, _):
            c0 = 2 * m
            pltpu.make_async_copy(eblk.at[0], ebufB, semEB).wait()
            pltpu.make_async_copy(msgB, hnext.at[dstidx.at[1]], semSB).wait()
            pltpu.async_copy(hsrc.at[ebufB.at[0]], msgB, semGB)
            pltpu.make_async_copy(hsrc.at[ebufA.at[0]], msgA, semGA).wait()
            _scale(msgA, ebufA, rk)
            _cp_dstidx(ebufA, 0)
            pltpu.async_copy(msgA, hnext.at[dstidx.at[0]], semSA, add=True)
            c2 = jnp.minimum(c0 + 2, NCHUNK - 1)
            pltpu.async_copy(eblk.at[c2], ebufA, semEA)
            pltpu.make_async_copy(eblk.at[0], ebufA, semEA).wait()
            pltpu.make_async_copy(msgA, hnext.at[dstidx.at[0]], semSA).wait()
            pltpu.async_copy(hsrc.at[ebufA.at[0]], msgA, semGA)
            pltpu.make_async_copy(hsrc.at[ebufB.at[0]], msgB, semGB).wait()
            _scale(msgB, ebufB, rk)
            _cp_dstidx(ebufB, 1)
            pltpu.async_copy(msgB, hnext.at[dstidx.at[1]], semSB, add=True)
            c3 = jnp.minimum(c0 + 3, NCHUNK - 1)
            pltpu.async_copy(eblk.at[c3], ebufB, semEB)
            return 0
        lax.fori_loop(0, NCHUNK // 2, _pair, 0)
        pltpu.make_async_copy(hsrc.at[ebufA.at[0]], msgA, semGA).wait()
        pltpu.make_async_copy(msgB, hnext.at[dstidx.at[1]], semSB).wait()
        pltpu.make_async_copy(eblk.at[0], ebufB, semEB).wait()
        plsc.subcore_barrier()

        for i in range(NRC):
            rs = pl.ds(rbase + i * W, W)
            pltpu.sync_copy(hnext.at[rs], msgA)
            pltpu.sync_copy(msgA, hw_hbm.at[cid, bdst, rs])
            pltpu.sync_copy(out_hbm.at[cid, rs], msgB)

            def _a(r, _):
                for j in range(C // 16):
                    sl = pl.ds(j * 16, 16)
                    msgB[r, sl] = msgB[r, sl] + msgA[r, sl]
                return 0
            lax.fori_loop(0, W, _a, 0)
            pltpu.sync_copy(msgB, out_hbm.at[cid, rs])

    def _pair(it, _):
        _hop(1, 0, 2 * it)
        _hop(0, 1, 2 * it + 1)
        return 0
    lax.fori_loop(0, K // 2, _pair, 0)


@functools.lru_cache(maxsize=None)
def _get_prop():
  return pl.kernel(
    _prop_body,
    out_type=(
        jax.ShapeDtypeStruct((2, NPAD, CW), jnp.float32),
        jax.ShapeDtypeStruct((2, 2, NPAD, CW), jnp.float32),
    ),
    mesh=plsc.VectorSubcoreMesh(core_axis_name="c", subcore_axis_name="s",
                                num_cores=2, num_subcores=NTILES),
    scratch_types=[
        pltpu.VMEM_SHARED((NPAD, CW), jnp.float32),
        pltpu.VMEM((3, W), jnp.int32),
        pltpu.VMEM((3, W), jnp.int32),
        pltpu.VMEM((2, W), jnp.int32),
        pltpu.VMEM((W, CW), jnp.float32),
        pltpu.VMEM((W, CW), jnp.float32),
        pltpu.VMEM((16,), jnp.float32),
        pltpu.SemaphoreType.DMA,
        pltpu.SemaphoreType.DMA,
        pltpu.SemaphoreType.DMA,
        pltpu.SemaphoreType.DMA,
        pltpu.SemaphoreType.DMA,
        pltpu.SemaphoreType.DMA,
    ],
  )


def _prep_edges(ei, nrm):
    src = ei[0].astype(jnp.int32).reshape(NTILES, EPT)
    dst = ei[1].astype(jnp.int32).reshape(NTILES, EPT)
    nr = nrm.astype(jnp.float32).reshape(NTILES, EPT)
    pad = EPT_PAD - EPT
    psrc = jnp.full((NTILES, pad), N - 2, jnp.int32)
    pdst = jnp.broadcast_to(N + (jnp.arange(pad, dtype=jnp.int32) % 240),
                            (NTILES, pad))
    pnrm = jnp.zeros((NTILES, pad), jnp.float32)
    src = jnp.concatenate([src, psrc], 1).reshape(NTILES, NCHUNK, W)
    dst = jnp.concatenate([dst, pdst], 1).reshape(NTILES, NCHUNK, W)
    nr = lax.bitcast_convert_type(
        jnp.concatenate([nr, pnrm], 1).reshape(NTILES, NCHUNK, W), jnp.int32)
    return jnp.stack([src, dst, nr], axis=2)


def kernel(feature, edge_index, edge_index2, norm_A, norm_A_2,
           W1, b1, W2, b2, temp1, temp2):
    feature_pad = jnp.pad(feature, ((0, NPAD - N), (0, 0)))
    x = _mlp(feature_pad, W1, b1, W2, b2)

    e1 = _prep_edges(edge_index, norm_A)
    e2 = _prep_edges(edge_index2, norm_A_2)
    edges = jnp.stack([e1, e2])
    temps = jnp.stack([jnp.pad(temp1, (0, 5)), jnp.pad(temp2, (0, 5))])

    out, _ = _get_prop()(x, edges, temps)
    return _final_add(out[0, :N, :C], out[1, :N, :C])


# speedup vs baseline: 1.1645x; 1.0004x over previous
import functools

import jax
import jax.numpy as jnp
from jax import lax
from jax.experimental import pallas as pl
from jax.experimental.pallas import tpu as pltpu
from jax.experimental.pallas import tpu_sc as plsc

N = 10000
NPAD = 10240
C = 64
CW = 128
K = 10
NTILES = 16
ROWS = NPAD // NTILES
E = 320000
EPT = E // NTILES
W = 128
EPT_PAD = 20480
NCHUNK = EPT_PAD // W
NRC = ROWS // W


def _mlp_body(f_ref, w1_ref, b1_ref, w2_ref, b2_ref, o_ref):
    h = jnp.dot(f_ref[...], w1_ref[...],
                preferred_element_type=jnp.float32) + b1_ref[...]
    h = jnp.maximum(h, 0.0)
    x = jnp.dot(h, w2_ref[...],
                preferred_element_type=jnp.float32) + b2_ref[...]
    o_ref[...] = jnp.concatenate(
        [x, jnp.zeros((x.shape[0], CW - C), jnp.float32)], axis=1)


def _mlp(feature_pad, W1, b1, W2, b2):
    nblk = NPAD // 1024
    return pl.pallas_call(
        _mlp_body,
        grid=(nblk,),
        in_specs=[
            pl.BlockSpec((1024, 128), lambda i: (i, 0)),
            pl.BlockSpec((128, 128), lambda i: (0, 0)),
            pl.BlockSpec((1, 128), lambda i: (0, 0)),
            pl.BlockSpec((128, C), lambda i: (0, 0)),
            pl.BlockSpec((1, C), lambda i: (0, 0)),
        ],
        out_specs=pl.BlockSpec((1024, CW), lambda i: (i, 0)),
        out_shape=jax.ShapeDtypeStruct((NPAD, CW), jnp.float32),
    )(feature_pad, W1, b1.reshape(1, 128), W2, b2.reshape(1, C))


def _add_body(a_ref, b_ref, o_ref):
    o_ref[...] = a_ref[...] + b_ref[...]


def _final_add(a, b):
    return pl.pallas_call(
        _add_body,
        grid=(10,),
        in_specs=[
            pl.BlockSpec((1000, C), lambda i: (i, 0)),
            pl.BlockSpec((1000, C), lambda i: (i, 0)),
        ],
        out_specs=pl.BlockSpec((1000, C), lambda i: (i, 0)),
        out_shape=jax.ShapeDtypeStruct((N, C), jnp.float32),
    )(a, b)


def _splat(vec16, idx):
    idxs = jnp.broadcast_to(jnp.asarray(idx, jnp.int32), (16,))
    return jnp.take_along_axis(vec16, idxs, axis=0)


def _prop_body(xw_hbm, edges_hbm, temps_hbm,
               out_hbm, hw_hbm,
               hnext, ebufA, ebufB, dstidx, msgA, msgB, temps_v,
               semGA, semGB, semSA, semSB):
    cid = lax.axis_index("c")
    sid = lax.axis_index("s")
    rbase = sid * ROWS

    pltpu.sync_copy(temps_hbm.at[cid], temps_v)
    t16 = temps_v[...]
    t0 = _splat(t16, 0)

    for i in range(NRC):
        rs = pl.ds(rbase + i * W, W)
        pltpu.sync_copy(xw_hbm.at[rs], msgA)

        def _s0(r, _):
            for j in range(C // 16):
                sl = pl.ds(j * 16, 16)
                msgA[r, sl] = msgA[r, sl] * t0
            return 0
        lax.fori_loop(0, W, _s0, 0)
        pltpu.sync_copy(msgA, hw_hbm.at[cid, 1, rs])
        pltpu.sync_copy(msgA, out_hbm.at[cid, rs])

    def _scale(mref, eref, rk):
        def _q(q, _):
            n16 = lax.bitcast_convert_type(
                eref[2, pl.ds(q * 16, 16)], jnp.float32) * rk
            for e in range(16):
                nb = jnp.broadcast_to(
                    lax.slice(n16, (e,), (e + 1,)), (16,))
                row = q * 16 + e
                for j in range(C // 16):
                    sl = pl.ds(j * 16, 16)
                    mref[row, sl] = mref[row, sl] * nb
            return 0
        lax.fori_loop(0, W // 16, _q, 0)

    def _cp_dstidx(eref, row):
        for j in range(W // 16):
            sl = pl.ds(j * 16, 16)
            dstidx[row, sl] = eref[1, sl]

    def _hop(bsrc, bdst, k):
        def _z(r, _):
            for j in range(CW // 16):
                sl = pl.ds(j * 16, 16)
                msgA[r, sl] = jnp.zeros((16,), jnp.float32)
                msgB[r, sl] = jnp.zeros((16,), jnp.float32)
            return 0
        lax.fori_loop(0, W, _z, 0)
        for i in range(NRC):
            pltpu.sync_copy(msgA, hnext.at[pl.ds(rbase + i * W, W)])
        plsc.subcore_barrier()

        rk = _splat(t16, k + 1) / _splat(t16, k)
        hsrc = hw_hbm.at[cid, bsrc]

        pltpu.sync_copy(edges_hbm.at[cid, sid, 0], ebufA)
        pltpu.async_copy(hsrc.at[ebufA.at[0]], msgA, semGA)
        _cp_dstidx(ebufA, 1)
        pltpu.async_copy(msgB, hnext.at[dstidx.at[1]], semSB, add=True)

        def _pair(m, _):
            c0 = 2 * m
            pltpu.sync_copy(edges_hbm.at[cid, sid, c0 + 1], ebufB)
            pltpu.make_async_copy(msgB, hnext.at[dstidx.at[1]], semSB).wait()
            pltpu.async_copy(hsrc.at[ebufB.at[0]], msgB, semGB)
            pltpu.make_async_copy(hsrc.at[ebufA.at[0]], msgA, semGA).wait()
            _scale(msgA, ebufA, rk)
            _cp_dstidx(ebufA, 0)
            pltpu.async_copy(msgA, hnext.at[dstidx.at[0]], semSA, add=True)
            c2 = jnp.minimum(c0 + 2, NCHUNK - 1)
            pltpu.sync_copy(edges_hbm.at[cid, sid, c2], ebufA)
            pltpu.make_async_copy(msgA, hnext.at[dstidx.at[0]], semSA).wait()
            pltpu.async_copy(hsrc.at[ebufA.at[0]], msgA, semGA)
            pltpu.make_async_copy(hsrc.at[ebufB.at[0]], msgB, semGB).wait()
            _scale(msgB, ebufB, rk)
            _cp_dstidx(ebufB, 1)
            pltpu.async_copy(msgB, hnext.at[dstidx.at[1]], semSB, add=True)
            return 0
        lax.fori_loop(0, NCHUNK // 2, _pair, 0)
        pltpu.make_async_copy(hsrc.at[ebufA.at[0]], msgA, semGA).wait()
        pltpu.make_async_copy(msgB, hnext.at[dstidx.at[1]], semSB).wait()
        plsc.subcore_barrier()

        for i in range(NRC):
            rs = pl.ds(rbase + i * W, W)
            pltpu.sync_copy(hnext.at[rs], msgA)
            pltpu.sync_copy(msgA, hw_hbm.at[cid, bdst, rs])
            pltpu.sync_copy(out_hbm.at[cid, rs], msgB)

            def _a(r, _):
                for j in range(C // 16):
                    sl = pl.ds(j * 16, 16)
                    msgB[r, sl] = msgB[r, sl] + msgA[r, sl]
                return 0
            lax.fori_loop(0, W, _a, 0)
            pltpu.sync_copy(msgB, out_hbm.at[cid, rs])

    def _pair(it, _):
        _hop(1, 0, 2 * it)
        _hop(0, 1, 2 * it + 1)
        return 0
    lax.fori_loop(0, K // 2, _pair, 0)


@functools.lru_cache(maxsize=None)
def _get_prop():
  return pl.kernel(
    _prop_body,
    out_type=(
        jax.ShapeDtypeStruct((2, NPAD, CW), jnp.float32),
        jax.ShapeDtypeStruct((2, 2, NPAD, CW), jnp.float32),
    ),
    mesh=plsc.VectorSubcoreMesh(core_axis_name="c", subcore_axis_name="s",
                                num_cores=2, num_subcores=NTILES),
    scratch_types=[
        pltpu.VMEM_SHARED((NPAD, CW), jnp.float32),
        pltpu.VMEM((3, W), jnp.int32),
        pltpu.VMEM((3, W), jnp.int32),
        pltpu.VMEM((2, W), jnp.int32),
        pltpu.VMEM((W, CW), jnp.float32),
        pltpu.VMEM((W, CW), jnp.float32),
        pltpu.VMEM((16,), jnp.float32),
        pltpu.SemaphoreType.DMA,
        pltpu.SemaphoreType.DMA,
        pltpu.SemaphoreType.DMA,
        pltpu.SemaphoreType.DMA,
    ],
  )


def _prep_edges(ei, nrm):
    src = ei[0].astype(jnp.int32).reshape(NTILES, EPT)
    dst = ei[1].astype(jnp.int32).reshape(NTILES, EPT)
    nr = nrm.astype(jnp.float32).reshape(NTILES, EPT)
    pad = EPT_PAD - EPT
    psrc = jnp.full((NTILES, pad), N - 2, jnp.int32)
    pdst = jnp.broadcast_to(N + (jnp.arange(pad, dtype=jnp.int32) % 240),
                            (NTILES, pad))
    pnrm = jnp.zeros((NTILES, pad), jnp.float32)
    src = jnp.concatenate([src, psrc], 1).reshape(NTILES, NCHUNK, W)
    dst = jnp.concatenate([dst, pdst], 1).reshape(NTILES, NCHUNK, W)
    nr = lax.bitcast_convert_type(
        jnp.concatenate([nr, pnrm], 1).reshape(NTILES, NCHUNK, W), jnp.int32)
    return jnp.stack([src, dst, nr], axis=2)


def kernel(feature, edge_index, edge_index2, norm_A, norm_A_2,
           W1, b1, W2, b2, temp1, temp2):
    feature_pad = jnp.pad(feature, ((0, NPAD - N), (0, 0)))
    x = _mlp(feature_pad, W1, b1, W2, b2)

    e1 = _prep_edges(edge_index, norm_A)
    e2 = _prep_edges(edge_index2, norm_A_2)
    edges = jnp.stack([e1, e2])
    temps = jnp.stack([jnp.pad(temp1, (0, 5)), jnp.pad(temp2, (0, 5))])

    out, _ = _get_prop()(x, edges, temps)
    return _final_add(out[0, :N, :C], out[1, :N, :C])
